# Initial kernel scaffold; baseline (speedup 1.0000x reference)
import jax
import jax.numpy as jnp
from jax.experimental import pallas as pl


def kernel(x, k):
    raise NotImplementedError("write your pallas kernel here")


# speedup vs baseline: 1.3338x; 1.3338x over previous
import jax
import jax.numpy as jnp
from jax.experimental import pallas as pl

K_BINS = 1024
EMB = 64
TB = 512


def _vq_block(x_ref, k_ref, kt_ref, xl_ref, xd_ref, part_ref):
    xb = x_ref[0]
    kmat = k_ref[...]
    ktmat = kt_ref[...]

    s = jax.lax.dot_general(
        kmat, xb, (((1,), (0,)), ((), ())),
        preferred_element_type=jnp.float32,
        precision=jax.lax.Precision.DEFAULT,
    )

    ksq = jnp.sum(kmat * kmat, axis=1, keepdims=True)
    xsq = jnp.sum(xb * xb, axis=0, keepdims=True)
    d = (xsq - 2.0 * s) + ksq

    mn = jnp.min(d, axis=0, keepdims=True)
    jidx = jax.lax.broadcasted_iota(jnp.int32, (K_BINS, TB), 0)
    idx = jnp.min(jnp.where(d == mn, jidx, K_BINS), axis=0)
    xl_ref[0, 0, :] = idx

    onehot = (jidx == idx[None, :]).astype(jnp.float32)
    xd = jax.lax.dot_general(
        ktmat, onehot, (((1,), (0,)), ((), ())),
        preferred_element_type=jnp.float32,
        precision=jax.lax.Precision.HIGHEST,
    )
    xd_ref[0] = xd

    lane = jax.lax.broadcasted_iota(jnp.int32, (1, 128), 1)
    sx = jnp.sum(xb)
    sq = jnp.sum(xsq)
    sm = jnp.sum(mn)
    part_ref[0] = jnp.where(
        lane == 0, sx, jnp.where(lane == 1, sq, jnp.where(lane == 2, sm, 0.0)))


def kernel(x, k):
    N, W, T = x.shape
    kt = jnp.transpose(k)
    nt = T // TB
    xl3, xd, parts = pl.pallas_call(
        _vq_block,
        grid=(N, nt),
        in_specs=[
            pl.BlockSpec((1, W, TB), lambda n, t: (n, 0, t)),
            pl.BlockSpec((K_BINS, EMB), lambda n, t: (0, 0)),
            pl.BlockSpec((EMB, K_BINS), lambda n, t: (0, 0)),
        ],
        out_specs=[
            pl.BlockSpec((1, 1, TB), lambda n, t: (n * nt + t, 0, 0)),
            pl.BlockSpec((1, W, TB), lambda n, t: (n, 0, t)),
            pl.BlockSpec((1, 1, 128), lambda n, t: (n * nt + t, 0, 0)),
        ],
        out_shape=[
            jax.ShapeDtypeStruct((N * nt, 1, TB), jnp.int32),
            jax.ShapeDtypeStruct((N, W, T), jnp.float32),
            jax.ShapeDtypeStruct((N * nt, 1, 128), jnp.float32),
        ],
    )(x, k, kt)

    x_l = xl3.reshape(N, T)
    sums = jnp.sum(parts[:, 0, :], axis=0)
    sx, sq, sm = sums[0], sums[1], sums[2]
    MD = jnp.float32(N * T * W)
    M = jnp.float32(N * T)
    fit = sm / M
    commit_loss = sm / MD
    prenorm = jnp.sqrt(jnp.maximum(sq - sx * sx / MD, 0.0) / MD)
    return (x_l, xd, commit_loss, fit, prenorm)


# speedup vs baseline: 2.0361x; 1.5265x over previous
import jax
import jax.numpy as jnp
from jax.experimental import pallas as pl

K_BINS = 1024
EMB = 64
TB = 512


def _vq_block(x_ref, k_ref, kth_ref, ktl_ref, xl_ref, xd_ref, part_ref):
    xb = x_ref[0]
    kmat = k_ref[...]

    s = jax.lax.dot_general(
        kmat, xb, (((1,), (0,)), ((), ())),
        preferred_element_type=jnp.float32,
        precision=jax.lax.Precision.DEFAULT,
    )

    ksq = jnp.sum(kmat * kmat, axis=1, keepdims=True)
    xsq = jnp.sum(xb * xb, axis=0, keepdims=True)
    d = (xsq - 2.0 * s) + ksq

    mn = jnp.min(d, axis=0, keepdims=True)
    jidx = jax.lax.broadcasted_iota(jnp.int32, (K_BINS, TB), 0)
    idx = jnp.min(jnp.where(d == mn, jidx, K_BINS), axis=0)
    xl_ref[0, 0, :] = idx

    onehot = (jidx == idx[None, :]).astype(jnp.bfloat16)
    dn = (((1,), (0,)), ((), ()))
    xd_hi = jax.lax.dot_general(kth_ref[...], onehot, dn,
                                preferred_element_type=jnp.float32)
    xd_lo = jax.lax.dot_general(ktl_ref[...], onehot, dn,
                                preferred_element_type=jnp.float32)
    xd_ref[0] = xd_hi + xd_lo

    lane = jax.lax.broadcasted_iota(jnp.int32, (1, 128), 1)
    sx = jnp.sum(xb)
    sq = jnp.sum(xsq)
    sm = jnp.sum(mn)
    part_ref[0] = jnp.where(
        lane == 0, sx, jnp.where(lane == 1, sq, jnp.where(lane == 2, sm, 0.0)))


def kernel(x, k):
    N, W, T = x.shape
    kt = jnp.transpose(k)
    kt_hi = kt.astype(jnp.bfloat16)
    kt_lo = (kt - kt_hi.astype(jnp.float32)).astype(jnp.bfloat16)
    nt = T // TB
    xl3, xd, parts = pl.pallas_call(
        _vq_block,
        grid=(N, nt),
        in_specs=[
            pl.BlockSpec((1, W, TB), lambda n, t: (n, 0, t)),
            pl.BlockSpec((K_BINS, EMB), lambda n, t: (0, 0)),
            pl.BlockSpec((EMB, K_BINS), lambda n, t: (0, 0)),
            pl.BlockSpec((EMB, K_BINS), lambda n, t: (0, 0)),
        ],
        out_specs=[
            pl.BlockSpec((1, 1, TB), lambda n, t: (n * nt + t, 0, 0)),
            pl.BlockSpec((1, W, TB), lambda n, t: (n, 0, t)),
            pl.BlockSpec((1, 1, 128), lambda n, t: (n * nt + t, 0, 0)),
        ],
        out_shape=[
            jax.ShapeDtypeStruct((N * nt, 1, TB), jnp.int32),
            jax.ShapeDtypeStruct((N, W, T), jnp.float32),
            jax.ShapeDtypeStruct((N * nt, 1, 128), jnp.float32),
        ],
    )(x, k, kt_hi, kt_lo)

    x_l = xl3.reshape(N, T)
    sums = jnp.sum(parts[:, 0, :], axis=0)
    sx, sq, sm = sums[0], sums[1], sums[2]
    MD = jnp.float32(N * T * W)
    M = jnp.float32(N * T)
    fit = sm / M
    commit_loss = sm / MD
    prenorm = jnp.sqrt(jnp.maximum(sq - sx * sx / MD, 0.0) / MD)
    return (x_l, xd, commit_loss, fit, prenorm)


# speedup vs baseline: 2.2561x; 1.1081x over previous
import jax
import jax.numpy as jnp
from jax.experimental import pallas as pl

K_BINS = 1024
EMB = 64
TB = 512


def _vq_block(x_ref, k_ref, kth_ref, ktl_ref, xl_ref, xd_ref, part_ref):
    xb = x_ref[0]
    kmat = k_ref[...]

    s = jax.lax.dot_general(
        kmat, xb, (((1,), (0,)), ((), ())),
        preferred_element_type=jnp.float32,
        precision=jax.lax.Precision.DEFAULT,
    )

    ksq = jnp.sum(kmat * kmat, axis=1, keepdims=True)
    xsq = jnp.sum(xb * xb, axis=0, keepdims=True)
    d = (xsq - 2.0 * s) + ksq

    mn = jnp.min(d, axis=0, keepdims=True)
    jidx = jax.lax.broadcasted_iota(jnp.int32, (K_BINS, TB), 0)
    idx = jnp.argmin(d, axis=0).astype(jnp.int32)
    xl_ref[0, 0, :] = idx

    onehot = (jidx == idx[None, :]).astype(jnp.bfloat16)
    dn = (((1,), (0,)), ((), ()))
    xd_hi = jax.lax.dot_general(kth_ref[...], onehot, dn,
                                preferred_element_type=jnp.float32)
    xd_lo = jax.lax.dot_general(ktl_ref[...], onehot, dn,
                                preferred_element_type=jnp.float32)
    xd_ref[0] = xd_hi + xd_lo

    lane = jax.lax.broadcasted_iota(jnp.int32, (1, 128), 1)
    sx = jnp.sum(xb)
    sq = jnp.sum(xsq)
    sm = jnp.sum(mn)
    part_ref[0] = jnp.where(
        lane == 0, sx, jnp.where(lane == 1, sq, jnp.where(lane == 2, sm, 0.0)))


def kernel(x, k):
    N, W, T = x.shape
    kt = jnp.transpose(k)
    kt_bits = jax.lax.bitcast_convert_type(kt, jnp.int32)
    kt_hif = jax.lax.bitcast_convert_type(
        jnp.bitwise_and(kt_bits, jnp.int32(-65536)), jnp.float32)
    kt_hi = kt_hif.astype(jnp.bfloat16)
    kt_lo = (kt - kt_hif).astype(jnp.bfloat16)
    nt = T // TB
    xl3, xd, parts = pl.pallas_call(
        _vq_block,
        grid=(N, nt),
        in_specs=[
            pl.BlockSpec((1, W, TB), lambda n, t: (n, 0, t)),
            pl.BlockSpec((K_BINS, EMB), lambda n, t: (0, 0)),
            pl.BlockSpec((EMB, K_BINS), lambda n, t: (0, 0)),
            pl.BlockSpec((EMB, K_BINS), lambda n, t: (0, 0)),
        ],
        out_specs=[
            pl.BlockSpec((1, 1, TB), lambda n, t: (n * nt + t, 0, 0)),
            pl.BlockSpec((1, W, TB), lambda n, t: (n, 0, t)),
            pl.BlockSpec((1, 1, 128), lambda n, t: (n * nt + t, 0, 0)),
        ],
        out_shape=[
            jax.ShapeDtypeStruct((N * nt, 1, TB), jnp.int32),
            jax.ShapeDtypeStruct((N, W, T), jnp.float32),
            jax.ShapeDtypeStruct((N * nt, 1, 128), jnp.float32),
        ],
    )(x, k, kt_hi, kt_lo)

    x_l = xl3.reshape(N, T)
    sums = jnp.sum(parts[:, 0, :], axis=0)
    sx, sq, sm = sums[0], sums[1], sums[2]
    MD = jnp.float32(N * T * W)
    M = jnp.float32(N * T)
    fit = sm / M
    commit_loss = sm / MD
    prenorm = jnp.sqrt(jnp.maximum(sq - sx * sx / MD, 0.0) / MD)
    return (x_l, xd, commit_loss, fit, prenorm)


# speedup vs baseline: 2.4402x; 1.0816x over previous
import jax
import jax.numpy as jnp
from jax.experimental import pallas as pl
from jax.experimental.pallas import tpu as pltpu

K_BINS = 1024
EMB = 64
TB = 1024
CH = 256
NC = K_BINS // CH


def _vq_block(x_ref, k_ref, kth_ref, ktl_ref, xl_ref, xd_ref, part_ref,
              idxs_ref):
    g = pl.program_id(0)
    b = jax.lax.rem(g, 2)
    dn = (((1,), (0,)), ((), ()))

    pidx = idxs_ref[pl.ds(1 - b, 1), 0:1, :][0]
    jidx = jax.lax.broadcasted_iota(jnp.int32, (CH, TB), 0)
    xd = None
    for c in range(NC):
        ohc = (jidx == (pidx - c * CH)).astype(jnp.bfloat16)
        h = jax.lax.dot_general(kth_ref[:, c * CH:(c + 1) * CH], ohc, dn,
                                preferred_element_type=jnp.float32)
        l = jax.lax.dot_general(ktl_ref[:, c * CH:(c + 1) * CH], ohc, dn,
                                preferred_element_type=jnp.float32)
        hl = h + l
        xd = hl if xd is None else xd + hl
    xd_ref[0] = xd

    xb = x_ref[0]
    xsq = jnp.sum(xb * xb, axis=0, keepdims=True)
    run_mn = None
    run_idx = None
    for c in range(NC):
        kc = k_ref[c * CH:(c + 1) * CH, :]
        sc = jax.lax.dot_general(
            kc, xb, dn,
            preferred_element_type=jnp.float32,
            precision=jax.lax.Precision.DEFAULT,
        )
        ksqc = jnp.sum(kc * kc, axis=1, keepdims=True)
        dc = (xsq - 2.0 * sc) + ksqc
        mnc = jnp.min(dc, axis=0, keepdims=True)
        idxc = jnp.argmin(dc, axis=0).astype(jnp.int32)[None, :] + c * CH
        if c == 0:
            run_mn, run_idx = mnc, idxc
        else:
            run_idx = jnp.where(mnc < run_mn, idxc, run_idx)
            run_mn = jnp.minimum(run_mn, mnc)
    xl_ref[0, 0, :] = run_idx[0]
    idxs_ref[pl.ds(b, 1), 0:1, :] = run_idx[None]

    lane = jax.lax.broadcasted_iota(jnp.int32, (1, 128), 1)
    sx = jnp.sum(xb)
    sq = jnp.sum(xsq)
    sm = jnp.sum(run_mn)
    part_ref[0] = jnp.where(
        lane == 0, sx, jnp.where(lane == 1, sq, jnp.where(lane == 2, sm, 0.0)))


def kernel(x, k):
    N, W, T = x.shape
    kt = jnp.transpose(k)
    kt_bits = jax.lax.bitcast_convert_type(kt, jnp.int32)
    kt_hif = jax.lax.bitcast_convert_type(
        jnp.bitwise_and(kt_bits, jnp.int32(-65536)), jnp.float32)
    kt_hi = kt_hif.astype(jnp.bfloat16)
    kt_lo = (kt - kt_hif).astype(jnp.bfloat16)
    nt = T // TB
    G = N * nt

    def cur(gi):
        gc = jnp.minimum(gi, G - 1)
        return gc

    xl3, xd, parts = pl.pallas_call(
        _vq_block,
        grid=(G + 1,),
        in_specs=[
            pl.BlockSpec((1, W, TB), lambda g: (cur(g) // nt, 0, cur(g) % nt)),
            pl.BlockSpec((K_BINS, EMB), lambda g: (0, 0)),
            pl.BlockSpec((EMB, K_BINS), lambda g: (0, 0)),
            pl.BlockSpec((EMB, K_BINS), lambda g: (0, 0)),
        ],
        out_specs=[
            pl.BlockSpec((1, 1, TB), lambda g: (cur(g), 0, 0)),
            pl.BlockSpec(
                (1, W, TB),
                lambda g: (jnp.maximum(g - 1, 0) // nt, 0,
                           jnp.maximum(g - 1, 0) % nt)),
            pl.BlockSpec((1, 1, 128), lambda g: (cur(g), 0, 0)),
        ],
        out_shape=[
            jax.ShapeDtypeStruct((G, 1, TB), jnp.int32),
            jax.ShapeDtypeStruct((N, W, T), jnp.float32),
            jax.ShapeDtypeStruct((G, 1, 128), jnp.float32),
        ],
        scratch_shapes=[pltpu.VMEM((2, 8, TB), jnp.int32)],
    )(x, k, kt_hi, kt_lo)

    x_l = xl3.reshape(N, T)
    sums = jnp.sum(parts[:, 0, :], axis=0)
    sx, sq, sm = sums[0], sums[1], sums[2]
    MD = jnp.float32(N * T * W)
    M = jnp.float32(N * T)
    fit = sm / M
    commit_loss = sm / MD
    prenorm = jnp.sqrt(jnp.maximum(sq - sx * sx / MD, 0.0) / MD)
    return (x_l, xd, commit_loss, fit, prenorm)


# speedup vs baseline: 2.5212x; 1.0332x over previous
import jax
import jax.numpy as jnp
from jax.experimental import pallas as pl
from jax.experimental.pallas import tpu as pltpu

K_BINS = 1024
EMB = 64
TB = 1024
SUB = 512
NS = TB // SUB
CH = 256
NC = K_BINS // CH


def _vq_block(x_ref, kn2_ref, kth_ref, ktl_ref, xl_ref, xd_ref, part_ref,
              idxs_ref):
    g = pl.program_id(0)
    b = jax.lax.rem(g, 2)
    dn = (((1,), (0,)), ((), ()))

    pidx = idxs_ref[pl.ds(1 - b, 1), 0:1, :][0]
    jidx = jax.lax.broadcasted_iota(jnp.int32, (CH, TB), 0).astype(jnp.bfloat16)
    one = jnp.ones((CH, TB), jnp.bfloat16)
    zero = jnp.zeros((CH, TB), jnp.bfloat16)
    xd = None
    for c in range(NC):
        tgt = (pidx - c * CH).astype(jnp.bfloat16)
        ohc = jnp.where(jidx == tgt, one, zero)
        h = jax.lax.dot_general(kth_ref[:, c * CH:(c + 1) * CH], ohc, dn,
                                preferred_element_type=jnp.float32)
        l = jax.lax.dot_general(ktl_ref[:, c * CH:(c + 1) * CH], ohc, dn,
                                preferred_element_type=jnp.float32)
        hl = h + l
        xd = hl if xd is None else xd + hl
    xd_ref[0] = xd

    xb = x_ref[0]
    sm = None
    sxa = None
    sqa = None
    for s in range(NS):
        xs = xb[:, s * SUB:(s + 1) * SUB]
        xsq = jnp.sum(xs * xs, axis=0, keepdims=True)
        run_mn = None
        run_idx = None
        for c in range(NC):
            kc = kn2_ref[c * CH:(c + 1) * CH, :]
            sc = jax.lax.dot_general(
                kc, xs, dn,
                preferred_element_type=jnp.float32,
                precision=jax.lax.Precision.DEFAULT,
            )
            ksqc = 0.25 * jnp.sum(kc * kc, axis=1, keepdims=True)
            dc = (xsq + sc) + ksqc
            mnc = jnp.min(dc, axis=0, keepdims=True)
            idxc = jnp.argmin(dc, axis=0).astype(jnp.int32)[None, :] + c * CH
            if c == 0:
                run_mn, run_idx = mnc, idxc
            else:
                run_idx = jnp.where(mnc < run_mn, idxc, run_idx)
                run_mn = jnp.minimum(run_mn, mnc)
        xl_ref[0, 0, s * SUB:(s + 1) * SUB] = run_idx[0]
        idxs_ref[pl.ds(b, 1), 0:1, s * SUB:(s + 1) * SUB] = run_idx[None]
        smc = jnp.sum(run_mn)
        sxc = jnp.sum(xs)
        sqc = jnp.sum(xsq)
        sm = smc if sm is None else sm + smc
        sxa = sxc if sxa is None else sxa + sxc
        sqa = sqc if sqa is None else sqa + sqc

    lane = jax.lax.broadcasted_iota(jnp.int32, (1, 128), 1)
    part_ref[0] = jnp.where(
        lane == 0, sxa, jnp.where(lane == 1, sqa, jnp.where(lane == 2, sm,
                                                            0.0)))


def kernel(x, k):
    N, W, T = x.shape
    kn2 = -2.0 * k
    kt = jnp.transpose(k)
    kt_bits = jax.lax.bitcast_convert_type(kt, jnp.int32)
    kt_hif = jax.lax.bitcast_convert_type(
        jnp.bitwise_and(kt_bits, jnp.int32(-65536)), jnp.float32)
    kt_hi = kt_hif.astype(jnp.bfloat16)
    kt_lo = (kt - kt_hif).astype(jnp.bfloat16)
    nt = T // TB
    G = N * nt

    def cur(gi):
        return jnp.minimum(gi, G - 1)

    xl3, xd, parts = pl.pallas_call(
        _vq_block,
        grid=(G + 1,),
        in_specs=[
            pl.BlockSpec((1, W, TB), lambda g: (cur(g) // nt, 0, cur(g) % nt)),
            pl.BlockSpec((K_BINS, EMB), lambda g: (0, 0)),
            pl.BlockSpec((EMB, K_BINS), lambda g: (0, 0)),
            pl.BlockSpec((EMB, K_BINS), lambda g: (0, 0)),
        ],
        out_specs=[
            pl.BlockSpec((1, 1, TB), lambda g: (cur(g), 0, 0)),
            pl.BlockSpec(
                (1, W, TB),
                lambda g: (jnp.maximum(g - 1, 0) // nt, 0,
                           jnp.maximum(g - 1, 0) % nt)),
            pl.BlockSpec((1, 1, 128), lambda g: (cur(g), 0, 0)),
        ],
        out_shape=[
            jax.ShapeDtypeStruct((G, 1, TB), jnp.int32),
            jax.ShapeDtypeStruct((N, W, T), jnp.float32),
            jax.ShapeDtypeStruct((G, 1, 128), jnp.float32),
        ],
        scratch_shapes=[pltpu.VMEM((2, 8, TB), jnp.int32)],
    )(x, kn2, kt_hi, kt_lo)

    x_l = xl3.reshape(N, T)
    sums = jnp.sum(parts[:, 0, :], axis=0)
    sx, sq, sm = sums[0], sums[1], sums[2]
    MD = jnp.float32(N * T * W)
    M = jnp.float32(N * T)
    fit = sm / M
    commit_loss = sm / MD
    prenorm = jnp.sqrt(jnp.maximum(sq - sx * sx / MD, 0.0) / MD)
    return (x_l, xd, commit_loss, fit, prenorm)


# speedup vs baseline: 2.6034x; 1.0326x over previous
import jax
import jax.numpy as jnp
from jax.experimental import pallas as pl
from jax.experimental.pallas import tpu as pltpu

K_BINS = 1024
EMB = 64
TB = 1024
SUB = 512
NS = TB // SUB
CH = 256
NC = K_BINS // CH
ACH = 1024
ANC = K_BINS // ACH


def _vq_block(x_ref, kn2_ref, kth_ref, ktl_ref, xl_ref, xd_ref, part_ref,
              idxs_ref):
    g = pl.program_id(0)
    b = jax.lax.rem(g, 2)
    dn = (((1,), (0,)), ((), ()))

    pidx = idxs_ref[pl.ds(1 - b, 1), 0:1, :][0]
    jidx = jax.lax.broadcasted_iota(jnp.int32, (CH, TB), 0).astype(jnp.bfloat16)
    one = jnp.ones((CH, TB), jnp.bfloat16)
    zero = jnp.zeros((CH, TB), jnp.bfloat16)
    oh = jnp.concatenate(
        [jnp.where(jidx == (pidx - c * CH).astype(jnp.bfloat16), one, zero)
         for c in range(NC)], axis=0)
    h = jax.lax.dot_general(kth_ref[...], oh, dn,
                            preferred_element_type=jnp.float32)
    l = jax.lax.dot_general(ktl_ref[...], oh, dn,
                            preferred_element_type=jnp.float32)
    xd_ref[0] = h + l

    xb = x_ref[0]
    sm = None
    sxa = None
    sqa = None
    for s in range(NS):
        xs = xb[:, s * SUB:(s + 1) * SUB]
        xsq = jnp.sum(xs * xs, axis=0, keepdims=True)
        run_mn = None
        run_idx = None
        for c in range(ANC):
            kc = kn2_ref[c * ACH:(c + 1) * ACH, :]
            sc = jax.lax.dot_general(
                kc, xs, dn,
                preferred_element_type=jnp.float32,
                precision=jax.lax.Precision.DEFAULT,
            )
            ksqc = 0.25 * jnp.sum(kc * kc, axis=1, keepdims=True)
            dc = (xsq + sc) + ksqc
            mnc = jnp.min(dc, axis=0, keepdims=True)
            idxc = jnp.argmin(dc, axis=0).astype(jnp.int32)[None, :] + c * ACH
            if c == 0:
                run_mn, run_idx = mnc, idxc
            else:
                run_idx = jnp.where(mnc < run_mn, idxc, run_idx)
                run_mn = jnp.minimum(run_mn, mnc)
        xl_ref[0, 0, s * SUB:(s + 1) * SUB] = run_idx[0]
        idxs_ref[pl.ds(b, 1), 0:1, s * SUB:(s + 1) * SUB] = run_idx[None]
        smc = jnp.sum(run_mn)
        sxc = jnp.sum(xs)
        sqc = jnp.sum(xsq)
        sm = smc if sm is None else sm + smc
        sxa = sxc if sxa is None else sxa + sxc
        sqa = sqc if sqa is None else sqa + sqc

    lane = jax.lax.broadcasted_iota(jnp.int32, (1, 128), 1)
    part_ref[0] = jnp.where(
        lane == 0, sxa, jnp.where(lane == 1, sqa, jnp.where(lane == 2, sm,
                                                            0.0)))


def kernel(x, k):
    N, W, T = x.shape
    kn2 = -2.0 * k
    kt = jnp.transpose(k)
    kt_bits = jax.lax.bitcast_convert_type(kt, jnp.int32)
    kt_hif = jax.lax.bitcast_convert_type(
        jnp.bitwise_and(kt_bits, jnp.int32(-65536)), jnp.float32)
    kt_hi = kt_hif.astype(jnp.bfloat16)
    kt_lo = (kt - kt_hif).astype(jnp.bfloat16)
    nt = T // TB
    G = N * nt

    def cur(gi):
        return jnp.minimum(gi, G - 1)

    xl3, xd, parts = pl.pallas_call(
        _vq_block,
        grid=(G + 1,),
        in_specs=[
            pl.BlockSpec((1, W, TB), lambda g: (cur(g) // nt, 0, cur(g) % nt)),
            pl.BlockSpec((K_BINS, EMB), lambda g: (0, 0)),
            pl.BlockSpec((EMB, K_BINS), lambda g: (0, 0)),
            pl.BlockSpec((EMB, K_BINS), lambda g: (0, 0)),
        ],
        out_specs=[
            pl.BlockSpec((1, 1, TB), lambda g: (cur(g), 0, 0)),
            pl.BlockSpec(
                (1, W, TB),
                lambda g: (jnp.maximum(g - 1, 0) // nt, 0,
                           jnp.maximum(g - 1, 0) % nt)),
            pl.BlockSpec((1, 1, 128), lambda g: (cur(g), 0, 0)),
        ],
        out_shape=[
            jax.ShapeDtypeStruct((G, 1, TB), jnp.int32),
            jax.ShapeDtypeStruct((N, W, T), jnp.float32),
            jax.ShapeDtypeStruct((G, 1, 128), jnp.float32),
        ],
        scratch_shapes=[pltpu.VMEM((2, 8, TB), jnp.int32)],
    )(x, kn2, kt_hi, kt_lo)

    x_l = xl3.reshape(N, T)
    sums = jnp.sum(parts[:, 0, :], axis=0)
    sx, sq, sm = sums[0], sums[1], sums[2]
    MD = jnp.float32(N * T * W)
    M = jnp.float32(N * T)
    fit = sm / M
    commit_loss = sm / MD
    prenorm = jnp.sqrt(jnp.maximum(sq - sx * sx / MD, 0.0) / MD)
    return (x_l, xd, commit_loss, fit, prenorm)
